# Initial kernel scaffold; baseline (speedup 1.0000x reference)
import jax
import jax.numpy as jnp
from jax.experimental import pallas as pl


def kernel(x, table, bias):
    raise NotImplementedError("write your pallas kernel here")


# speedup vs baseline: 1.2304x; 1.2304x over previous
import jax
import jax.numpy as jnp
from jax import lax
from jax.experimental import pallas as pl
from jax.experimental.pallas import tpu as pltpu
from jax.experimental.pallas import tpu_sc as plsc

NUM_FIELDS = 26
FIELD_DIM = 100000
BATCH = 16384
L = 16
NC, NS = 2, 16
NW = NC * NS
B_PER_W = BATCH // NW
N_PER_W = B_PER_W * NUM_FIELDS
CHUNK = 128
N_CHUNKS = N_PER_W // CHUNK


def _body(xt_hbm, tab_hbm, bias_hbm, out_hbm, idx_v, rows_v, out_v, bias_v, sem):
    wid = lax.axis_index("c") * NS + lax.axis_index("s")

    for f in range(NUM_FIELDS):
        pltpu.make_async_copy(
            xt_hbm.at[pl.ds(f * BATCH + wid * B_PER_W, B_PER_W)],
            idx_v.at[pl.ds(f * B_PER_W, B_PER_W)],
            sem,
        ).start()
    pltpu.sync_copy(bias_hbm, bias_v)
    for f in range(NUM_FIELDS):
        pltpu.make_async_copy(
            xt_hbm.at[pl.ds(f * BATCH + wid * B_PER_W, B_PER_W)],
            idx_v.at[pl.ds(f * B_PER_W, B_PER_W)],
            sem,
        ).wait()

    def add_offsets(f, carry):
        off = f * FIELD_DIM

        def inner(c, carry2):
            o = f * B_PER_W + c * L
            idx_v[pl.ds(o, L)] = idx_v[pl.ds(o, L)] + off
            return carry2

        return lax.fori_loop(0, B_PER_W // L, inner, carry)

    lax.fori_loop(0, NUM_FIELDS, add_offsets, 0)

    def fire(j, carry):
        o = j * CHUNK
        pltpu.make_async_copy(
            tab_hbm.at[idx_v.at[pl.ds(o, CHUNK)]],
            rows_v.at[pl.ds(o, CHUNK)],
            sem,
        ).start()
        return carry

    lax.fori_loop(0, N_CHUNKS, fire, 0)

    def drain(j, carry):
        o = j * CHUNK
        pltpu.make_async_copy(
            tab_hbm.at[idx_v.at[pl.ds(o, CHUNK)]],
            rows_v.at[pl.ds(o, CHUNK)],
            sem,
        ).wait()
        return carry

    lax.fori_loop(0, N_CHUNKS, drain, 0)

    def reduce(c, carry):
        o = c * L
        acc = bias_v[...]
        for f in range(NUM_FIELDS):
            acc = acc + rows_v[pl.ds(f * B_PER_W + o, L)]
        out_v[pl.ds(o, L)] = acc
        return carry

    lax.fori_loop(0, B_PER_W // L, reduce, 0)

    pltpu.sync_copy(out_v, out_hbm.at[pl.ds(wid * B_PER_W, B_PER_W)])


@jax.jit
def _run(xt_flat, tab_flat, bias16):
    mesh = plsc.VectorSubcoreMesh(core_axis_name="c", subcore_axis_name="s")
    return pl.kernel(
        _body,
        out_type=jax.ShapeDtypeStruct((BATCH,), jnp.float32),
        mesh=mesh,
        scratch_types=[
            pltpu.VMEM((N_PER_W,), jnp.int32),
            pltpu.VMEM((N_PER_W,), jnp.float32),
            pltpu.VMEM((B_PER_W,), jnp.float32),
            pltpu.VMEM((L,), jnp.float32),
            pltpu.SemaphoreType.DMA,
        ],
    )(xt_flat, tab_flat, bias16)


def kernel(x, table, bias):
    xt_flat = x.T.reshape(-1)
    tab_flat = table.reshape(-1)
    bias16 = jnp.broadcast_to(bias, (L,))
    out = _run(xt_flat, tab_flat, bias16)
    return out.reshape(BATCH, 1)
